# Initial kernel scaffold; baseline (speedup 1.0000x reference)
import jax
import jax.numpy as jnp
from jax.experimental import pallas as pl


def kernel(hidden_states, router_w, gate_w, up_w, down_w, shared_gate_w, shared_up_w, shared_down_w):
    raise NotImplementedError("write your pallas kernel here")


# speedup vs baseline: 1.0550x; 1.0550x over previous
import functools

import jax
import jax.numpy as jnp
from jax import lax
from jax.experimental import pallas as pl
from jax.experimental.pallas import tpu as pltpu
from jax.experimental.pallas import tpu_sc as plsc

T, D, E, TK, I = 2048, 1024, 8, 2, 1024
BLK = 256
TB = T // BLK
NB_R = (T * TK) // BLK + E
NP_R = NB_R * BLK
NW = 32
GCHUNK = 64
CH4 = 32

_f32 = jnp.float32


def _router_body(h_ref, rw_ref, logits_ref, i12_ref, hs_ref):
    k = pl.program_id(0)
    x = h_ref[...]
    l = lax.dot_general(x, rw_ref[...], (((1,), (1,)), ((), ())),
                        preferred_element_type=_f32)
    iota_e = lax.broadcasted_iota(jnp.int32, (BLK, E), 1)
    m1 = jnp.max(l, axis=1, keepdims=True)
    i1 = jnp.min(jnp.where(l == m1, iota_e, E), axis=1, keepdims=True)
    l2 = jnp.where(iota_e == i1, -jnp.inf, l)
    m2 = jnp.max(l2, axis=1, keepdims=True)
    i2 = jnp.min(jnp.where(l2 == m2, iota_e, E), axis=1, keepdims=True)
    logits_ref[...] = l
    i12_ref[...] = jnp.where(iota_e == 0, i1, jnp.where(iota_e == 1, i2, 0))
    mv = jnp.where(k == 0, m1, m2)
    hs_ref[...] = x * (1.0 / (1.0 + jnp.exp(-mv)))


def _router(h2, router_w):
    return pl.pallas_call(
        _router_body,
        grid=(TK, TB),
        in_specs=[
            pl.BlockSpec((BLK, D), lambda k, i: (i, 0)),
            pl.BlockSpec((E, D), lambda k, i: (0, 0)),
        ],
        out_specs=[
            pl.BlockSpec((BLK, E), lambda k, i: (i, 0)),
            pl.BlockSpec((BLK, E), lambda k, i: (i, 0)),
            pl.BlockSpec((BLK, D), lambda k, i: (k * TB + i, 0)),
        ],
        out_shape=[
            jax.ShapeDtypeStruct((T, E), _f32),
            jax.ShapeDtypeStruct((T, E), jnp.int32),
            jax.ShapeDtypeStruct((TK * T, D), _f32),
        ],
    )(h2, router_w)


def _metadata(i12):
    e_flat = jnp.concatenate([i12[:, 0], i12[:, 1]])
    oh = jax.nn.one_hot(e_flat, E, dtype=jnp.int32)
    counts = jnp.sum(oh, axis=0)
    ranks = jnp.cumsum(oh, axis=0) - oh
    padded = ((counts + BLK - 1) // BLK) * BLK
    cum_pad = jnp.cumsum(padded)
    starts = cum_pad - padded
    dest = starts[e_flat] + jnp.sum(ranks * oh, axis=1)
    flat_idx = jnp.zeros((NP_R,), jnp.int32).at[dest].set(
        jnp.arange(TK * T, dtype=jnp.int32))
    pos0, pos1 = dest[:T], dest[T:]
    bid = jnp.arange(NB_R, dtype=jnp.int32)
    be = jnp.minimum(
        jnp.searchsorted(cum_pad, bid * BLK, side="right"), E - 1
    ).astype(jnp.int32)
    ba = (bid * BLK < starts[be] + counts[be]).astype(jnp.int32)
    return flat_idx, pos0, pos1, be, ba


@functools.cache
def _sc_mesh():
    return plsc.VectorSubcoreMesh(core_axis_name="c", subcore_axis_name="s")


@functools.cache
def _sc_dispatch_kernel():
    @functools.partial(
        pl.kernel,
        mesh=_sc_mesh(),
        out_type=jax.ShapeDtypeStruct((NP_R, D), _f32),
        scratch_types=[
            pltpu.VMEM((GCHUNK,), jnp.int32),
            pltpu.VMEM((GCHUNK, D), _f32),
            pltpu.SemaphoreType.DMA,
        ],
    )
    def body(hs_hbm, idx_hbm, out_hbm, idx_v, rows_v, sem):
        w = lax.axis_index("s") * 2 + lax.axis_index("c")
        base = w * (NP_R // NW)
        for c in range(NP_R // NW // GCHUNK):
            off = base + c * GCHUNK
            pltpu.sync_copy(idx_hbm.at[pl.ds(off, GCHUNK)], idx_v)
            pltpu.async_copy(hs_hbm.at[idx_v], rows_v, sem).wait()
            pltpu.sync_copy(rows_v, out_hbm.at[pl.ds(off, GCHUNK)])

    return body


def _sc_dispatch(hs, flat_idx):
    return _sc_dispatch_kernel()(hs, flat_idx)


def _moe_mm_body(be_ref, ba_ref, x_ref, gw_ref, uw_ref, dw_ref, o_ref):
    b = pl.program_id(0)

    @pl.when(ba_ref[b] != 0)
    def _():
        x = x_ref[...]
        g = lax.dot_general(x, gw_ref[0], (((1,), (1,)), ((), ())),
                            preferred_element_type=_f32)
        u = lax.dot_general(x, uw_ref[0], (((1,), (1,)), ((), ())),
                            preferred_element_type=_f32)
        a = g * (1.0 / (1.0 + jnp.exp(-g))) * u
        o_ref[...] = lax.dot_general(a, dw_ref[0], (((1,), (1,)), ((), ())),
                                     preferred_element_type=_f32)


def _moe_mm(be, ba, x_sorted, gate_w, up_w, down_w):
    grid_spec = pltpu.PrefetchScalarGridSpec(
        num_scalar_prefetch=2,
        grid=(NB_R,),
        in_specs=[
            pl.BlockSpec((BLK, D), lambda b, be, ba: (b, 0)),
            pl.BlockSpec((1, I, D), lambda b, be, ba: (be[b], 0, 0)),
            pl.BlockSpec((1, I, D), lambda b, be, ba: (be[b], 0, 0)),
            pl.BlockSpec((1, D, I), lambda b, be, ba: (be[b], 0, 0)),
        ],
        out_specs=pl.BlockSpec((BLK, D), lambda b, be, ba: (b, 0)),
    )
    return pl.pallas_call(
        _moe_mm_body,
        grid_spec=grid_spec,
        out_shape=jax.ShapeDtypeStruct((NP_R, D), _f32),
    )(be, ba, x_sorted, gate_w, up_w, down_w)


@functools.cache
def _sc_return_kernel():
    @functools.partial(
        pl.kernel,
        mesh=_sc_mesh(),
        out_type=(
            jax.ShapeDtypeStruct((T, D), _f32),
            jax.ShapeDtypeStruct((T, D), _f32),
        ),
        scratch_types=[
            pltpu.VMEM((CH4,), jnp.int32),
            pltpu.VMEM((CH4, D), _f32),
            pltpu.SemaphoreType.DMA,
        ],
    )
    def body(o_hbm, p0_hbm, p1_hbm, o0_hbm, o1_hbm, idx_v, rows_v, sem):
        w = lax.axis_index("s") * 2 + lax.axis_index("c")
        base = w * (T // NW)
        for src, dst in ((p0_hbm, o0_hbm), (p1_hbm, o1_hbm)):
            for c in range(T // NW // CH4):
                off = base + c * CH4
                pltpu.sync_copy(src.at[pl.ds(off, CH4)], idx_v)
                pltpu.async_copy(o_hbm.at[idx_v], rows_v, sem).wait()
                pltpu.sync_copy(rows_v, dst.at[pl.ds(off, CH4)])

    return body


def _sc_return(o, pos0, pos1):
    return _sc_return_kernel()(o, pos0, pos1)


def _shared_body(h_ref, gw_ref, uw_ref, dw_ref, o0_ref, o1_ref, out_ref):
    x = h_ref[...]
    g = lax.dot_general(x, gw_ref[...], (((1,), (1,)), ((), ())),
                        preferred_element_type=_f32)
    u = lax.dot_general(x, uw_ref[...], (((1,), (1,)), ((), ())),
                        preferred_element_type=_f32)
    a = g * (1.0 / (1.0 + jnp.exp(-g))) * u
    sh = lax.dot_general(a, dw_ref[...], (((1,), (1,)), ((), ())),
                         preferred_element_type=_f32)
    out_ref[...] = sh + o0_ref[...] + o1_ref[...]


def _shared_combine(h2, sgw, suw, sdw, o0, o1):
    return pl.pallas_call(
        _shared_body,
        grid=(TB,),
        in_specs=[
            pl.BlockSpec((BLK, D), lambda i: (i, 0)),
            pl.BlockSpec((I, D), lambda i: (0, 0)),
            pl.BlockSpec((I, D), lambda i: (0, 0)),
            pl.BlockSpec((D, I), lambda i: (0, 0)),
            pl.BlockSpec((BLK, D), lambda i: (i, 0)),
            pl.BlockSpec((BLK, D), lambda i: (i, 0)),
        ],
        out_specs=pl.BlockSpec((BLK, D), lambda i: (i, 0)),
        out_shape=jax.ShapeDtypeStruct((T, D), _f32),
    )(h2, sgw, suw, sdw, o0, o1)


def kernel(hidden_states, router_w, gate_w, up_w, down_w,
           shared_gate_w, shared_up_w, shared_down_w):
    b, s, d = hidden_states.shape
    h2 = hidden_states.reshape(T, D)
    logits, i12, hs = _router(h2, router_w)
    flat_idx, pos0, pos1, be, ba = _metadata(i12)
    x_sorted = _sc_dispatch(hs, flat_idx)
    o = _moe_mm(be, ba, x_sorted, gate_w, up_w, down_w)
    o0, o1 = _sc_return(o, pos0, pos1)
    out = _shared_combine(h2, shared_gate_w, shared_up_w, shared_down_w, o0, o1)
    return out.reshape(b, s, d), logits.reshape(b, s, E)


# speedup vs baseline: 1.6091x; 1.5252x over previous
import functools

import jax
import jax.numpy as jnp
from jax import lax
from jax.experimental import pallas as pl
from jax.experimental.pallas import tpu as pltpu
from jax.experimental.pallas import tpu_sc as plsc

T, D, E, TK, I = 2048, 1024, 8, 2, 1024
BLK = 256
TB = T // BLK
NB_R = (T * TK) // BLK + E
NP_R = NB_R * BLK
NW = 32
GCHUNK = 48
CH4 = 32

_f32 = jnp.float32


def _router_body(h_ref, rw_ref, logits_ref, i12_ref, hs_ref):
    k = pl.program_id(0)
    x = h_ref[...]
    l = lax.dot_general(x, rw_ref[...], (((1,), (1,)), ((), ())),
                        preferred_element_type=_f32)
    iota_e = lax.broadcasted_iota(jnp.int32, (BLK, E), 1)
    m1 = jnp.max(l, axis=1, keepdims=True)
    i1 = jnp.min(jnp.where(l == m1, iota_e, E), axis=1, keepdims=True)
    l2 = jnp.where(iota_e == i1, -jnp.inf, l)
    m2 = jnp.max(l2, axis=1, keepdims=True)
    i2 = jnp.min(jnp.where(l2 == m2, iota_e, E), axis=1, keepdims=True)
    logits_ref[...] = l
    i12_ref[...] = jnp.where(iota_e == 0, i1, jnp.where(iota_e == 1, i2, 0))
    mv = jnp.where(k == 0, m1, m2)
    hs_ref[...] = x * (1.0 / (1.0 + jnp.exp(-mv)))


def _router(h2, router_w):
    return pl.pallas_call(
        _router_body,
        grid=(TK, TB),
        in_specs=[
            pl.BlockSpec((BLK, D), lambda k, i: (i, 0)),
            pl.BlockSpec((E, D), lambda k, i: (0, 0)),
        ],
        out_specs=[
            pl.BlockSpec((BLK, E), lambda k, i: (i, 0)),
            pl.BlockSpec((BLK, E), lambda k, i: (i, 0)),
            pl.BlockSpec((BLK, D), lambda k, i: (k * TB + i, 0)),
        ],
        out_shape=[
            jax.ShapeDtypeStruct((T, E), _f32),
            jax.ShapeDtypeStruct((T, E), jnp.int32),
            jax.ShapeDtypeStruct((TK * T, D), _f32),
        ],
    )(h2, router_w)


def _metadata(i12):
    e_flat = jnp.concatenate([i12[:, 0], i12[:, 1]])
    oh = jax.nn.one_hot(e_flat, E, dtype=jnp.int32)
    counts = jnp.sum(oh, axis=0)
    ranks = jnp.cumsum(oh, axis=0) - oh
    padded = ((counts + BLK - 1) // BLK) * BLK
    cum_pad = jnp.cumsum(padded)
    starts = cum_pad - padded
    dest = starts[e_flat] + jnp.sum(ranks * oh, axis=1)
    pad_fill = jnp.arange(NP_R, dtype=jnp.int32) % (TK * T)
    flat_idx = pad_fill.at[dest].set(jnp.arange(TK * T, dtype=jnp.int32))
    pos0, pos1 = dest[:T], dest[T:]
    bid = jnp.arange(NB_R, dtype=jnp.int32)
    be = jnp.minimum(
        jnp.searchsorted(cum_pad, bid * BLK, side="right"), E - 1
    ).astype(jnp.int32)
    ba = (bid * BLK < starts[be] + counts[be]).astype(jnp.int32)
    return flat_idx, pos0, pos1, be, ba


@functools.cache
def _sc_mesh():
    return plsc.VectorSubcoreMesh(core_axis_name="c", subcore_axis_name="s")


def _pipelined_gather(table_hbm, jobs, bufs):
    n = len(jobs)
    copies = [None, None]
    for c in range(n):
        iv, rv, sm = bufs[c % 2]
        idx_src, _ = jobs[c]
        pltpu.sync_copy(idx_src, iv)
        copies[c % 2] = pltpu.async_copy(table_hbm.at[iv], rv, sm)
        if c > 0:
            _, out_dst = jobs[c - 1]
            copies[(c - 1) % 2].wait()
            pltpu.sync_copy(bufs[(c - 1) % 2][1], out_dst)
    copies[(n - 1) % 2].wait()
    pltpu.sync_copy(bufs[(n - 1) % 2][1], jobs[n - 1][1])


@functools.cache
def _sc_dispatch_kernel():
    @functools.partial(
        pl.kernel,
        mesh=_sc_mesh(),
        out_type=jax.ShapeDtypeStruct((NP_R, D), _f32),
        scratch_types=[
            pltpu.VMEM((GCHUNK,), jnp.int32),
            pltpu.VMEM((GCHUNK, D), _f32),
            pltpu.SemaphoreType.DMA,
            pltpu.VMEM((GCHUNK,), jnp.int32),
            pltpu.VMEM((GCHUNK, D), _f32),
            pltpu.SemaphoreType.DMA,
        ],
    )
    def body(hs_hbm, idx_hbm, out_hbm, ia, ra, sa, ib, rb, sb):
        w = lax.axis_index("s") * 2 + lax.axis_index("c")
        base = w * (NP_R // NW)
        jobs = []
        for c in range(NP_R // NW // GCHUNK):
            off = base + c * GCHUNK
            jobs.append((idx_hbm.at[pl.ds(off, GCHUNK)],
                         out_hbm.at[pl.ds(off, GCHUNK)]))
        _pipelined_gather(hs_hbm, jobs, ((ia, ra, sa), (ib, rb, sb)))

    return body


def _sc_dispatch(hs, flat_idx):
    return _sc_dispatch_kernel()(hs, flat_idx)


def _moe_mm_body(be_ref, ba_ref, x_ref, gw_ref, uw_ref, dw_ref, o_ref):
    b = pl.program_id(0)

    @pl.when(ba_ref[b] != 0)
    def _():
        x = x_ref[...]
        g = lax.dot_general(x, gw_ref[0], (((1,), (1,)), ((), ())),
                            preferred_element_type=_f32)
        u = lax.dot_general(x, uw_ref[0], (((1,), (1,)), ((), ())),
                            preferred_element_type=_f32)
        a = g * (1.0 / (1.0 + jnp.exp(-g))) * u
        o_ref[...] = lax.dot_general(a, dw_ref[0], (((1,), (1,)), ((), ())),
                                     preferred_element_type=_f32)


def _moe_mm(be, ba, x_sorted, gate_w, up_w, down_w):
    grid_spec = pltpu.PrefetchScalarGridSpec(
        num_scalar_prefetch=2,
        grid=(NB_R,),
        in_specs=[
            pl.BlockSpec((BLK, D), lambda b, be, ba: (b, 0)),
            pl.BlockSpec((1, I, D), lambda b, be, ba: (be[b], 0, 0)),
            pl.BlockSpec((1, I, D), lambda b, be, ba: (be[b], 0, 0)),
            pl.BlockSpec((1, D, I), lambda b, be, ba: (be[b], 0, 0)),
        ],
        out_specs=pl.BlockSpec((BLK, D), lambda b, be, ba: (b, 0)),
    )
    return pl.pallas_call(
        _moe_mm_body,
        grid_spec=grid_spec,
        out_shape=jax.ShapeDtypeStruct((NP_R, D), _f32),
    )(be, ba, x_sorted, gate_w, up_w, down_w)


@functools.cache
def _sc_return_kernel():
    @functools.partial(
        pl.kernel,
        mesh=_sc_mesh(),
        out_type=(
            jax.ShapeDtypeStruct((T, D), _f32),
            jax.ShapeDtypeStruct((T, D), _f32),
        ),
        scratch_types=[
            pltpu.VMEM((CH4,), jnp.int32),
            pltpu.VMEM((CH4, D), _f32),
            pltpu.SemaphoreType.DMA,
            pltpu.VMEM((CH4,), jnp.int32),
            pltpu.VMEM((CH4, D), _f32),
            pltpu.SemaphoreType.DMA,
        ],
    )
    def body(o_hbm, p0_hbm, p1_hbm, o0_hbm, o1_hbm, ia, ra, sa, ib, rb, sb):
        w = lax.axis_index("s") * 2 + lax.axis_index("c")
        base = w * (T // NW)
        jobs = []
        for src, dst in ((p0_hbm, o0_hbm), (p1_hbm, o1_hbm)):
            for c in range(T // NW // CH4):
                off = base + c * CH4
                jobs.append((src.at[pl.ds(off, CH4)],
                             dst.at[pl.ds(off, CH4)]))
        _pipelined_gather(o_hbm, jobs, ((ia, ra, sa), (ib, rb, sb)))

    return body


def _sc_return(o, pos0, pos1):
    return _sc_return_kernel()(o, pos0, pos1)


def _shared_body(h_ref, gw_ref, uw_ref, dw_ref, o0_ref, o1_ref, out_ref):
    x = h_ref[...]
    g = lax.dot_general(x, gw_ref[...], (((1,), (1,)), ((), ())),
                        preferred_element_type=_f32)
    u = lax.dot_general(x, uw_ref[...], (((1,), (1,)), ((), ())),
                        preferred_element_type=_f32)
    a = g * (1.0 / (1.0 + jnp.exp(-g))) * u
    sh = lax.dot_general(a, dw_ref[...], (((1,), (1,)), ((), ())),
                         preferred_element_type=_f32)
    out_ref[...] = sh + o0_ref[...] + o1_ref[...]


def _shared_combine(h2, sgw, suw, sdw, o0, o1):
    return pl.pallas_call(
        _shared_body,
        grid=(TB,),
        in_specs=[
            pl.BlockSpec((BLK, D), lambda i: (i, 0)),
            pl.BlockSpec((I, D), lambda i: (0, 0)),
            pl.BlockSpec((I, D), lambda i: (0, 0)),
            pl.BlockSpec((D, I), lambda i: (0, 0)),
            pl.BlockSpec((BLK, D), lambda i: (i, 0)),
            pl.BlockSpec((BLK, D), lambda i: (i, 0)),
        ],
        out_specs=pl.BlockSpec((BLK, D), lambda i: (i, 0)),
        out_shape=jax.ShapeDtypeStruct((T, D), _f32),
    )(h2, sgw, suw, sdw, o0, o1)


def kernel(hidden_states, router_w, gate_w, up_w, down_w,
           shared_gate_w, shared_up_w, shared_down_w):
    b, s, d = hidden_states.shape
    h2 = hidden_states.reshape(T, D)
    logits, i12, hs = _router(h2, router_w)
    flat_idx, pos0, pos1, be, ba = _metadata(i12)
    x_sorted = _sc_dispatch(hs, flat_idx)
    o = _moe_mm(be, ba, x_sorted, gate_w, up_w, down_w)
    o0, o1 = _sc_return(o, pos0, pos1)
    out = _shared_combine(h2, shared_gate_w, shared_up_w, shared_down_w, o0, o1)
    return out.reshape(b, s, d), logits.reshape(b, s, E)
